# baseline (speedup 1.0000x reference)
import functools

import jax
import jax.numpy as jnp
from jax import lax
from jax.experimental import pallas as pl
from jax.experimental.pallas import tpu as pltpu
from jax.experimental.pallas import tpu_sc as plsc

D_MODEL = 1024
N_IDX = 32768

_info = plsc.get_sparse_core_info()
_NC = _info.num_cores
_NS = _info.num_subcores
_NW = _NC * _NS
B_PER_W = N_IDX // _NW
CHUNK = 32
N_CHUNKS = B_PER_W // CHUNK

_mesh = plsc.VectorSubcoreMesh(core_axis_name="c", subcore_axis_name="s")


NBUF = 3


@functools.partial(
    pl.kernel,
    mesh=_mesh,
    out_type=jax.ShapeDtypeStruct((N_IDX, D_MODEL), jnp.float32),
    scratch_types=[
        pltpu.VMEM((B_PER_W,), jnp.int32),
        *[pltpu.VMEM((CHUNK, D_MODEL), jnp.float32) for _ in range(NBUF)],
        *[pltpu.SemaphoreType.DMA for _ in range(2 * NBUF)],
    ],
)
def _gather_kernel(idx_hbm, pe_hbm, out_hbm, idx_v, *scratch):
    bufs = scratch[:NBUF]
    gsems = scratch[NBUF : 2 * NBUF]
    wsems = scratch[2 * NBUF :]

    wid = lax.axis_index("s") * _NC + lax.axis_index("c")
    base = wid * B_PER_W
    pltpu.sync_copy(idx_hbm.at[pl.ds(base, B_PER_W)], idx_v)

    def start_gather(g, b):
        pltpu.async_copy(
            pe_hbm.at[idx_v.at[pl.ds(g * CHUNK, CHUNK)]], bufs[b], gsems[b]
        )

    def wait_gather(b):
        pltpu.make_async_copy(
            pe_hbm.at[idx_v.at[pl.ds(0, CHUNK)]], bufs[b], gsems[b]
        ).wait()

    def start_write(g, b):
        pltpu.async_copy(
            bufs[b], out_hbm.at[pl.ds(base + g * CHUNK, CHUNK)], wsems[b]
        )

    def wait_write(b):
        pltpu.make_async_copy(
            pe_hbm.at[idx_v.at[pl.ds(0, CHUNK)]], bufs[b], wsems[b]
        ).wait()

    start_gather(0, 0)

    @pl.loop(0, N_CHUNKS, step=NBUF)
    def _(g0):
        for b in range(NBUF):
            g = g0 + b
            nb = (b + 1) % NBUF

            @pl.when(jnp.logical_and(g + 1 < N_CHUNKS, g >= NBUF - 1))
            def _():
                wait_write(nb)

            @pl.when(g + 1 < N_CHUNKS)
            def _():
                start_gather(g + 1, nb)

            @pl.when(g < N_CHUNKS)
            def _():
                wait_gather(b)
                start_write(g, b)

    for g in range(N_CHUNKS - NBUF + 1, N_CHUNKS + 1):
        wait_write(g % NBUF)


def kernel(x, pe):
    return _gather_kernel(x.reshape(-1), pe)


# speedup vs baseline: 1.0003x; 1.0003x over previous
import functools

import jax
import jax.numpy as jnp
from jax import lax
from jax.experimental import pallas as pl
from jax.experimental.pallas import tpu as pltpu
from jax.experimental.pallas import tpu_sc as plsc

D_MODEL = 1024
N_IDX = 32768

_info = plsc.get_sparse_core_info()
_NC = _info.num_cores
_NS = _info.num_subcores
_NW = _NC * _NS
B_PER_W = N_IDX // _NW
CHUNK = 16
N_CHUNKS = B_PER_W // CHUNK

_mesh = plsc.VectorSubcoreMesh(core_axis_name="c", subcore_axis_name="s")


NBUF = 6


@functools.partial(
    pl.kernel,
    mesh=_mesh,
    out_type=jax.ShapeDtypeStruct((N_IDX, D_MODEL), jnp.float32),
    scratch_types=[
        pltpu.VMEM((B_PER_W,), jnp.int32),
        *[pltpu.VMEM((CHUNK, D_MODEL), jnp.float32) for _ in range(NBUF)],
        *[pltpu.SemaphoreType.DMA for _ in range(2 * NBUF)],
    ],
)
def _gather_kernel(idx_hbm, pe_hbm, out_hbm, idx_v, *scratch):
    bufs = scratch[:NBUF]
    gsems = scratch[NBUF : 2 * NBUF]
    wsems = scratch[2 * NBUF :]

    wid = lax.axis_index("s") * _NC + lax.axis_index("c")
    base = wid * B_PER_W
    pltpu.sync_copy(idx_hbm.at[pl.ds(base, B_PER_W)], idx_v)

    def start_gather(g, b):
        pltpu.async_copy(
            pe_hbm.at[idx_v.at[pl.ds(g * CHUNK, CHUNK)]], bufs[b], gsems[b]
        )

    def wait_gather(b):
        pltpu.make_async_copy(
            pe_hbm.at[idx_v.at[pl.ds(0, CHUNK)]], bufs[b], gsems[b]
        ).wait()

    def start_write(g, b):
        pltpu.async_copy(
            bufs[b], out_hbm.at[pl.ds(base + g * CHUNK, CHUNK)], wsems[b]
        )

    def wait_write(b):
        pltpu.make_async_copy(
            pe_hbm.at[idx_v.at[pl.ds(0, CHUNK)]], bufs[b], wsems[b]
        ).wait()

    start_gather(0, 0)

    @pl.loop(0, N_CHUNKS, step=NBUF)
    def _(g0):
        for b in range(NBUF):
            g = g0 + b
            nb = (b + 1) % NBUF

            @pl.when(jnp.logical_and(g + 1 < N_CHUNKS, g >= NBUF - 1))
            def _():
                wait_write(nb)

            @pl.when(g + 1 < N_CHUNKS)
            def _():
                start_gather(g + 1, nb)

            @pl.when(g < N_CHUNKS)
            def _():
                wait_gather(b)
                start_write(g, b)

    for g in range(N_CHUNKS - NBUF + 1, N_CHUNKS + 1):
        wait_write(g % NBUF)


def kernel(x, pe):
    return _gather_kernel(x.reshape(-1), pe)
